# Initial kernel scaffold; baseline (speedup 1.0000x reference)
import jax
import jax.numpy as jnp
from jax.experimental import pallas as pl


def kernel(y_1, y_2, t, epoch):
    raise NotImplementedError("write your pallas kernel here")


# speedup vs baseline: 3.4352x; 3.4352x over previous
import functools

import jax
import jax.numpy as jnp
from jax.experimental import pallas as pl
from jax.experimental.pallas import tpu as pltpu

_EPOCHS = 200
_DECAY_W = 1.0
_TH = 0.8
_INCREMENT = 0.5 / _EPOCHS
_CO_LAMBDA = 0.1

_R = 256


def _phase1_kernel(y1_ref, y2_ref, t_ref, loss_ref, dc_ref, cond_ref, s_ref):
    y1 = y1_ref[...]
    y2 = y2_ref[...]
    t = t_ref[0]
    C = y1.shape[1]

    m1 = jnp.max(y1, axis=1, keepdims=True)
    m2 = jnp.max(y2, axis=1, keepdims=True)
    e1 = jnp.exp(y1 - m1)
    e2 = jnp.exp(y2 - m2)
    s1 = jnp.sum(e1, axis=1, keepdims=True)
    s2 = jnp.sum(e2, axis=1, keepdims=True)
    lse1 = m1 + jnp.log(s1)
    lse2 = m2 + jnp.log(s2)

    iota = jax.lax.broadcasted_iota(jnp.int32, y1.shape, 1)
    onehot_t = iota == t
    y1t = jnp.sum(jnp.where(onehot_t, y1, 0.0), axis=1, keepdims=True)
    y2t = jnp.sum(jnp.where(onehot_t, y2, 0.0), axis=1, keepdims=True)

    a1 = jnp.min(jnp.where(y1 == m1, iota, C), axis=1, keepdims=True)
    a2 = jnp.min(jnp.where(y2 == m2, iota, C), axis=1, keepdims=True)
    y2a1 = jnp.sum(jnp.where(iota == a1, y2, 0.0), axis=1, keepdims=True)

    loss = (lse1 - y1t) + (lse2 - y2t)
    dc = (lse1 - m1) + (lse2 - y2a1)

    pmax_prod = (1.0 / s1) * (1.0 / s2)
    cond = jnp.logical_and(
        jnp.logical_and(a1 != t, a1 == a2), pmax_prod > _TH * _TH
    )

    q_diff = e1 / s1 - e2 / s2
    s_row = jnp.sum(q_diff * (y1 - y2), axis=1, keepdims=True)

    loss_ref[0] = loss.T
    dc_ref[0] = dc.T
    cond_ref[0] = cond.astype(jnp.float32).T
    s_ref[0] = s_row.T


def _phase2_kernel(loss_ref, dc_ref, cond_ref, s_ref, kfloor_ref, out_ref):
    loss = loss_ref[...]
    RR, CC = loss.shape
    n = RR * CC
    n_f = jnp.float32(n)

    sum_loss = jnp.sum(loss)
    mean_v = sum_loss / n_f
    n_small = jnp.sum((loss < mean_v).astype(jnp.int32))
    k = jnp.maximum(kfloor_ref[0, 0], n_small)

    bits = jax.lax.bitcast_convert_type(loss, jnp.int32)

    def body(_, carry):
        lo, hi = carry
        mid = lo + (hi - lo) // 2
        c = jnp.sum((bits <= mid).astype(jnp.int32))
        pred = c >= k
        return jnp.where(pred, lo, mid + 1), jnp.where(pred, mid, hi)

    _, vbits = jax.lax.fori_loop(
        0, 31, body, (jnp.int32(0), jnp.int32(0x7F800000))
    )

    count_less = jnp.sum((bits < vbits).astype(jnp.int32))
    eq = bits == vbits
    need = (k - count_less).astype(jnp.float32)

    eqf = eq.astype(jnp.float32)
    r0 = jax.lax.broadcasted_iota(jnp.int32, (RR, CC), 0)
    c1 = jax.lax.broadcasted_iota(jnp.int32, (RR, CC), 1)
    lower_incl = (r0 <= c1).astype(jnp.float32)
    strict_lower = (c1 < r0).astype(jnp.float32)
    incl = jnp.dot(eqf, lower_incl, preferred_element_type=jnp.float32)
    rowtot = incl[:, CC - 1 : CC]
    offs = jnp.dot(strict_lower, rowtot, preferred_element_type=jnp.float32)
    eq_before = incl - eqf + offs

    in_update = (bits < vbits) | (eq & (eq_before < need))

    flat_idx = r0 * CC + c1
    upd1 = jnp.logical_and(jnp.logical_not(in_update), flat_idx >= 1)
    condb = cond_ref[...] > 0.5

    loss_clean = jnp.sum(jnp.where(in_update, loss, 0.0))
    loss_dc = jnp.sum(jnp.where(jnp.logical_and(upd1, condb), dc_ref[...], 0.0))
    loss1 = jnp.sum(
        jnp.where(jnp.logical_and(upd1, jnp.logical_not(condb)), loss, 0.0)
    )
    inter = jnp.sum(s_ref[...])

    out_ref[0, 0] = (
        loss_clean + loss_dc + _DECAY_W * loss1
    ) / n_f + _CO_LAMBDA * (inter / n_f)


@functools.partial(jax.jit, static_argnames=())
def kernel(y_1, y_2, t, epoch):
    N, C = y_1.shape
    G = N // _R

    t3 = t.reshape(G, _R, 1)
    row_shape = jax.ShapeDtypeStruct((G, 1, _R), jnp.float32)
    loss, dc, cond, s = pl.pallas_call(
        _phase1_kernel,
        grid=(G,),
        in_specs=[
            pl.BlockSpec((_R, C), lambda i: (i, 0)),
            pl.BlockSpec((_R, C), lambda i: (i, 0)),
            pl.BlockSpec((1, _R, 1), lambda i: (i, 0, 0)),
        ],
        out_specs=[pl.BlockSpec((1, 1, _R), lambda i: (i, 0, 0))] * 4,
        out_shape=[row_shape] * 4,
    )(y_1, y_2, t3)

    RR = 128
    CC = N // RR
    loss2 = loss.reshape(RR, CC)
    dc2 = dc.reshape(RR, CC)
    cond2 = cond.reshape(RR, CC)
    s2 = s.reshape(RR, CC)

    remember_rate = 1.0 - _INCREMENT * epoch
    kfloor = jnp.floor(remember_rate * N).astype(jnp.int32).reshape(1, 1)

    out = pl.pallas_call(
        _phase2_kernel,
        in_specs=[
            pl.BlockSpec((RR, CC), lambda: (0, 0)),
            pl.BlockSpec((RR, CC), lambda: (0, 0)),
            pl.BlockSpec((RR, CC), lambda: (0, 0)),
            pl.BlockSpec((RR, CC), lambda: (0, 0)),
            pl.BlockSpec(memory_space=pltpu.SMEM),
        ],
        out_specs=pl.BlockSpec(memory_space=pltpu.SMEM),
        out_shape=jax.ShapeDtypeStruct((1, 1), jnp.float32),
    )(loss2, dc2, cond2, s2, kfloor)

    return out.reshape(())
